# Initial kernel scaffold; baseline (speedup 1.0000x reference)
import jax
import jax.numpy as jnp
from jax.experimental import pallas as pl


def kernel(x, edge_index, edge_attr, type, params):
    raise NotImplementedError("write your pallas kernel here")


# speedup vs baseline: 2.0987x; 2.0987x over previous
import functools

import jax
import jax.numpy as jnp
from jax import lax
from jax.experimental import pallas as pl
from jax.experimental.pallas import tpu as pltpu
from jax.experimental.pallas import tpu_sc as plsc

_N = 10000
_E = 320000
_LAT = 64
_EPS = 1e-5

_NC, _NS = 2, 16
_NW = _NC * _NS
_EPW = _E // _NW
_CH = 80
_NCH = _EPW // _CH
_NP = 10240
_NPT = _NP // _NS

_BN = 2048
_GN = 5
_BE = 4000


_STG = 320


def _sc_gather(tbl, dst3, src3):
    mesh = plsc.VectorSubcoreMesh(core_axis_name="c", subcore_axis_name="s")

    @functools.partial(
        pl.kernel, mesh=mesh,
        out_type=jax.ShapeDtypeStruct((_E, _LAT), jnp.float32),
        scratch_types=[pltpu.VMEM((_NCH, _CH), jnp.int32),
                       pltpu.VMEM((_NCH, _CH), jnp.int32),
                       pltpu.VMEM((_CH, 2 * _LAT), jnp.float32),
                       pltpu.VMEM((_CH, 2 * _LAT), jnp.float32),
                       pltpu.VMEM((_CH, _LAT), jnp.float32),
                       pltpu.SemaphoreType.DMA,
                       pltpu.SemaphoreType.DMA],
    )
    def k(t_h, d_h, s_h, g_h, di_v, si_v, rd_v, rs_v, g_v, sd, ss):
        sid = lax.axis_index("s")
        cid = lax.axis_index("c")
        wid = sid * _NC + cid
        pltpu.sync_copy(d_h.at[wid], di_v)
        pltpu.sync_copy(s_h.at[wid], si_v)

        base = wid * _EPW

        def body(j, carry):
            cd = pltpu.async_copy(t_h.at[di_v.at[j]], rd_v, sd)
            cs = pltpu.async_copy(t_h.at[si_v.at[j]], rs_v, ss)
            cd.wait()
            cs.wait()

            def add_row(r, carry2):
                for c in range(_LAT // 16):
                    g_v[r, pl.ds(c * 16, 16)] = (
                        rd_v[r, pl.ds(c * 16, 16)]
                        + rs_v[r, pl.ds(_LAT + c * 16, 16)])
                return carry2

            lax.fori_loop(0, _CH, add_row, 0, unroll=4)
            pltpu.sync_copy(g_v, g_h.at[pl.ds(base + j * _CH, _CH)])
            return carry

        lax.fori_loop(0, _NCH, body, 0, unroll=False)

    return k(tbl, dst3, src3)


def _sc_scatter(msg, dst3, zeros):
    mesh = plsc.VectorSubcoreMesh(core_axis_name="c", subcore_axis_name="s")

    @functools.partial(
        pl.kernel, mesh=mesh,
        out_type=jax.ShapeDtypeStruct((_NC, _NP, 2 * _LAT), jnp.float32),
        scratch_types=[pltpu.VMEM((_CH,), jnp.int32),
                       pltpu.VMEM((_CH, 2 * _LAT), jnp.float32),
                       pltpu.VMEM((_NPT // 4, 2 * _LAT), jnp.float32),
                       pltpu.VMEM_SHARED((_NP, 2 * _LAT), jnp.float32)],
    )
    def k(m_h, d_h, z_h, out_h, di_v, rows_v, big_v, acc_sh):
        cid = lax.axis_index("c")
        sid = lax.axis_index("s")
        wid = sid * _NC + cid
        for q in range(4):
            r0 = sid * _NPT + q * (_NPT // 4)
            pltpu.sync_copy(z_h.at[pl.ds(r0, _NPT // 4)], big_v)
            pltpu.sync_copy(big_v, acc_sh.at[pl.ds(r0, _NPT // 4)])
        plsc.subcore_barrier()

        base = wid * _EPW

        def body(j, carry):
            pltpu.sync_copy(d_h.at[wid, j], di_v)
            pltpu.sync_copy(m_h.at[pl.ds(base + j * _CH, _CH)], rows_v)
            pltpu.sync_copy(rows_v, acc_sh.at[di_v], add=True)
            return carry

        lax.fori_loop(0, _NCH, body, 0, unroll=False)
        plsc.subcore_barrier()
        for q in range(4):
            r0 = sid * _NPT + q * (_NPT // 4)
            pltpu.sync_copy(acc_sh.at[pl.ds(r0, _NPT // 4)], big_v)
            pltpu.sync_copy(big_v, out_h.at[cid, pl.ds(r0, _NPT // 4)])

    return k(msg, dst3, zeros)


def _dot(a, b):
    return jnp.dot(a, b, preferred_element_type=jnp.float32)


def _ln(x, g, b):
    m = jnp.mean(x, axis=-1, keepdims=True)
    v = jnp.mean((x - m) ** 2, axis=-1, keepdims=True)
    return (x - m) * lax.rsqrt(v + _EPS) * g + b


def _sp(x):
    return jnp.maximum(x, 0.0) + jnp.log1p(jnp.exp(-jnp.abs(x)))


def _row(i):
    return (i, 0)


def _rep(i):
    return (0, 0)


def _wspecs(ws):
    return [pl.BlockSpec(w.shape, _rep) for w in ws]


def _enc_node(x, typ, ws):
    def body(x_r, t_r, w0, b0, w1, b1, w2, b2, g, bl, w1d, bm1, w1s,
             q0, c0, q1, c1, q2, c2, n_o, tb_o, gm_o):
        h = jnp.maximum(_dot(x_r[...], w0[...]) + b0[...], 0.0)
        h = jnp.maximum(_dot(h, w1[...]) + b1[...], 0.0)
        n = _ln(_dot(h, w2[...]) + b2[...], g[...], bl[...])
        n_o[...] = n
        tb_o[...] = jnp.concatenate(
            [_dot(n, w1d[...]) + bm1[...], _dot(n, w1s[...])], axis=1)
        t = _sp(_dot(t_r[...], q0[...]) + c0[...])
        t = _sp(_dot(t, q1[...]) + c1[...])
        gm_o[...] = _sp(_dot(t, q2[...]) + c2[...])

    return pl.pallas_call(
        body,
        grid=(_GN,),
        in_specs=[pl.BlockSpec((_BN, x.shape[1]), _row),
                  pl.BlockSpec((_BN, typ.shape[1]), _row)] + _wspecs(ws),
        out_specs=[pl.BlockSpec((_BN, _LAT), _row),
                   pl.BlockSpec((_BN, 2 * _LAT), _row),
                   pl.BlockSpec((_BN, 1), _row)],
        out_shape=[jax.ShapeDtypeStruct((_N, _LAT), jnp.float32),
                   jax.ShapeDtypeStruct((_NP, 2 * _LAT), jnp.float32),
                   jax.ShapeDtypeStruct((_N, 1), jnp.float32)],
    )(x, typ, *ws)


def _enc_edge(ea, ws):
    grid = _E // _BE

    def body(e_r, w0, b0, w1, b1, w2, b2, g, bl, out_r):
        h = jnp.maximum(_dot(e_r[...], w0[...]) + b0[...], 0.0)
        h = jnp.maximum(_dot(h, w1[...]) + b1[...], 0.0)
        out_r[...] = _ln(_dot(h, w2[...]) + b2[...], g[...], bl[...])

    return pl.pallas_call(
        body,
        grid=(grid,),
        in_specs=[pl.BlockSpec((_BE, ea.shape[1]), _row)] + _wspecs(ws),
        out_specs=pl.BlockSpec((_BE, _LAT), _row),
        out_shape=jax.ShapeDtypeStruct((_E, _LAT), jnp.float32),
    )(ea, *ws)


def _edge_step(gsum, edge, ws):
    grid = _E // _BE

    def body(g_r, e_r, w1e, w2, b2, w3, b3, g, bl, msg_o, eo_o):
        e = e_r[...]
        h = jnp.maximum(g_r[...] + _dot(e, w1e[...]), 0.0)
        h = jnp.maximum(_dot(h, w2[...]) + b2[...], 0.0)
        msg = _ln(_dot(h, w3[...]) + b3[...], g[...], bl[...])
        msg_o[...] = jnp.concatenate([msg, jnp.zeros_like(msg)], axis=1)
        eo_o[...] = e + msg

    return pl.pallas_call(
        body,
        grid=(grid,),
        in_specs=[pl.BlockSpec((_BE, _LAT), _row)] * 2 + _wspecs(ws),
        out_specs=[pl.BlockSpec((_BE, 2 * _LAT), _row),
                   pl.BlockSpec((_BE, _LAT), _row)],
        out_shape=[jax.ShapeDtypeStruct((_E, 2 * _LAT), jnp.float32),
                   jax.ShapeDtypeStruct((_E, _LAT), jnp.float32)],
    )(gsum, edge, *ws)


def _node_body(n_r, a_r, u1n, u1a, b1, u2, b2, u3, b3, g, bl):
    n = n_r[...]
    a = a_r[...]
    agg = a[0, :, :_LAT] + a[1, :, :_LAT]
    h = jnp.maximum(_dot(n, u1n[...]) + _dot(agg, u1a[...]) + b1[...], 0.0)
    h = jnp.maximum(_dot(h, u2[...]) + b2[...], 0.0)
    upd = _ln(_dot(h, u3[...]) + b3[...], g[...], bl[...])
    return n + upd


_AGG_SPEC = pl.BlockSpec((2, _BN, 2 * _LAT), lambda i: (0, i, 0))


def _node_step(node, aggp, ws):
    def body(n_r, a_r, u1n, u1a, b1, u2, b2, u3, b3, g, bl,
             w1d, bm1, w1s, n_o, tb_o):
        nn = _node_body(n_r, a_r, u1n, u1a, b1, u2, b2, u3, b3, g, bl)
        n_o[...] = nn
        tb_o[...] = jnp.concatenate(
            [_dot(nn, w1d[...]) + bm1[...], _dot(nn, w1s[...])], axis=1)

    return pl.pallas_call(
        body,
        grid=(_GN,),
        in_specs=[pl.BlockSpec((_BN, _LAT), _row), _AGG_SPEC] + _wspecs(ws),
        out_specs=[pl.BlockSpec((_BN, _LAT), _row),
                   pl.BlockSpec((_BN, 2 * _LAT), _row)],
        out_shape=[jax.ShapeDtypeStruct((_N, _LAT), jnp.float32),
                   jax.ShapeDtypeStruct((_NP, 2 * _LAT), jnp.float32)],
    )(node, aggp, *ws)


def _decoder(node, ws):
    def body(n_r, d0, e0, d1, e1, d2, e2, m0, c0, m1, c1, m2, c2, f_o):
        o = jnp.maximum(_dot(n_r[...], d0[...]) + e0[...], 0.0)
        o = jnp.maximum(_dot(o, d1[...]) + e1[...], 0.0)
        o = _dot(o, d2[...]) + e2[...]
        f = _sp(_dot(o, m0[...]) + c0[...])
        f = _sp(_dot(f, m1[...]) + c1[...])
        f_o[...] = _dot(f, m2[...]) + c2[...]

    return pl.pallas_call(
        body,
        grid=(_GN,),
        in_specs=[pl.BlockSpec((_BN, _LAT), _row)] + _wspecs(ws),
        out_specs=pl.BlockSpec((_BN, 3), _row),
        out_shape=jax.ShapeDtypeStruct((_N, 3), jnp.float32),
    )(node, *ws)


def _b(v):
    return v.reshape(1, -1)


def _mlp_ws(ps):
    out = []
    for w, b in ps:
        out.extend([w, _b(b)])
    return out


def _split_w1(st):
    w1, b1 = st["edge_fn"][0]
    return w1[:_LAT], _b(b1), w1[_LAT:2 * _LAT], w1[2 * _LAT:]


def kernel(x, edge_index, edge_attr, type, params):
    p = params
    src = edge_index[0].astype(jnp.int32)
    dst = edge_index[1].astype(jnp.int32)
    dst3 = dst.reshape(_NW, _NCH, _CH)
    src3 = src.reshape(_NW, _NCH, _CH)
    zeros = jnp.zeros((_NP, 2 * _LAT), jnp.float32)

    steps = p["steps"]
    w1d0, b10, w1s0, _ = _split_w1(steps[0])

    enc_node_ws = (_mlp_ws(p["node_enc"])
                   + [_b(p["node_enc_ln"][0]), _b(p["node_enc_ln"][1])]
                   + [w1d0, b10, w1s0]
                   + _mlp_ws(p["mlp2"]))
    node, tbl, gamma = _enc_node(x, type, enc_node_ws)

    enc_edge_ws = (_mlp_ws(p["edge_enc"])
                   + [_b(p["edge_enc_ln"][0]), _b(p["edge_enc_ln"][1])])
    edge = _enc_edge(edge_attr, enc_edge_ws)

    def per_step(t):
        st = steps[t]
        w1d, bm1, w1s, w1e = _split_w1(st)
        del w1d, bm1, w1s
        ew = ([w1e] + _mlp_ws(st["edge_fn"][1:])
              + [_b(st["edge_ln"][0]), _b(st["edge_ln"][1])])
        un, bn = st["node_fn"][0]
        nw = ([un[:_LAT], un[_LAT:], _b(bn)] + _mlp_ws(st["node_fn"][1:])
              + [_b(st["node_ln"][0]), _b(st["node_ln"][1])])
        nst = steps[(t + 1) % len(steps)]
        w1dn, b1n, w1sn, _ = _split_w1(nst)
        return ew + nw + [w1dn, b1n, w1sn]

    n_ew = 7
    cols = list(zip(*[per_step(t) for t in range(len(steps))]))
    xs = [jnp.stack(c) for c in cols]

    def step_fn(carry, ws):
        node, tbl, edge = carry
        gsum = _sc_gather(tbl, dst3, src3)
        msg, edge = _edge_step(gsum, edge, ws[:n_ew])
        aggp = _sc_scatter(msg, dst3, zeros)
        node, tbl = _node_step(node, aggp, ws[n_ew:])
        return (node, tbl, edge), None

    (node, tbl, edge), _ = lax.scan(step_fn, (node, tbl, edge), xs)

    force = _decoder(node, _mlp_ws(p["decoder"]) + _mlp_ws(p["mlp1"]))
    return (force, gamma)


# speedup vs baseline: 2.4867x; 1.1849x over previous
import functools

import jax
import jax.numpy as jnp
from jax import lax
from jax.experimental import pallas as pl
from jax.experimental.pallas import tpu as pltpu
from jax.experimental.pallas import tpu_sc as plsc

_N = 10000
_E = 320000
_LAT = 64
_EPS = 1e-5

_NC, _NS = 2, 16
_NW = _NC * _NS
_EPW = _E // _NW
_CH = 80
_NCH = _EPW // _CH
_NP = 10240
_NPT = _NP // _NS

_BN = 2048
_GN = 5
_BE = 4000


_STG = 320


def _sc_gather(tbl, dst3, src3):
    mesh = plsc.VectorSubcoreMesh(core_axis_name="c", subcore_axis_name="s")

    @functools.partial(
        pl.kernel, mesh=mesh,
        out_type=jax.ShapeDtypeStruct((_E, _LAT), jnp.float32),
        scratch_types=[pltpu.VMEM((_NCH, _CH), jnp.int32),
                       pltpu.VMEM((_NCH, _CH), jnp.int32),
                       pltpu.VMEM((_CH, 2 * _LAT), jnp.float32),
                       pltpu.VMEM((_CH, 2 * _LAT), jnp.float32),
                       pltpu.VMEM((_CH, 2 * _LAT), jnp.float32),
                       pltpu.VMEM((_CH, 2 * _LAT), jnp.float32),
                       pltpu.VMEM((_CH, _LAT), jnp.float32),
                       pltpu.VMEM((_CH, _LAT), jnp.float32),
                       pltpu.SemaphoreType.DMA,
                       pltpu.SemaphoreType.DMA,
                       pltpu.SemaphoreType.DMA,
                       pltpu.SemaphoreType.DMA,
                       pltpu.SemaphoreType.DMA],
    )
    def k(t_h, d_h, s_h, g_h, di_v, si_v, rd0, rs0, rd1, rs1, g0, g1,
          sd0, ss0, sd1, ss1, sw):
        sid = lax.axis_index("s")
        cid = lax.axis_index("c")
        wid = sid * _NC + cid
        pltpu.sync_copy(d_h.at[wid], di_v)
        pltpu.sync_copy(s_h.at[wid], si_v)

        base = wid * _EPW
        rd = (rd0, rd1)
        rs = (rs0, rs1)
        gg = (g0, g1)
        sd = (sd0, sd1)
        ss = (ss0, ss1)

        def add_rows(rd_v, rs_v, g_v):
            def add_row(r, carry2):
                for c in range(_LAT // 16):
                    g_v[r, pl.ds(c * 16, 16)] = (
                        rd_v[r, pl.ds(c * 16, 16)]
                        + rs_v[r, pl.ds(_LAT + c * 16, 16)])
                return carry2

            lax.fori_loop(0, _CH, add_row, 0, unroll=8)

        def do_chunk(j, b, cpd, cps):
            cpd.wait()
            cps.wait()
            add_rows(rd[b], rs[b], gg[b])
            return pltpu.async_copy(
                gg[b], g_h.at[pl.ds(base + j * _CH, _CH)], sw)

        def outer(o, carry):
            j = o * 2
            cd0 = pltpu.async_copy(t_h.at[di_v.at[j]], rd[0], sd[0])
            cs0 = pltpu.async_copy(t_h.at[si_v.at[j]], rs[0], ss[0])
            cd1 = pltpu.async_copy(t_h.at[di_v.at[j + 1]], rd[1], sd[1])
            cs1 = pltpu.async_copy(t_h.at[si_v.at[j + 1]], rs[1], ss[1])
            w0 = do_chunk(j, 0, cd0, cs0)
            w1 = do_chunk(j + 1, 1, cd1, cs1)
            w0.wait()
            w1.wait()
            return carry

        lax.fori_loop(0, _NCH // 2, outer, 0, unroll=False)
        jl = _NCH - 1
        cd = pltpu.async_copy(t_h.at[di_v.at[jl]], rd[0], sd[0])
        cs = pltpu.async_copy(t_h.at[si_v.at[jl]], rs[0], ss[0])
        do_chunk(jl, 0, cd, cs).wait()

    return k(tbl, dst3, src3)


def _sc_scatter(msg, dst3, zeros):
    mesh = plsc.VectorSubcoreMesh(core_axis_name="c", subcore_axis_name="s")

    @functools.partial(
        pl.kernel, mesh=mesh,
        out_type=jax.ShapeDtypeStruct((_NC, _NP, 2 * _LAT), jnp.float32),
        scratch_types=[pltpu.VMEM((_NCH, _CH), jnp.int32),
                       pltpu.VMEM((_CH, 2 * _LAT), jnp.float32),
                       pltpu.VMEM((_CH, 2 * _LAT), jnp.float32),
                       pltpu.VMEM((_CH, 2 * _LAT), jnp.float32),
                       pltpu.VMEM_SHARED((_NP, 2 * _LAT), jnp.float32),
                       pltpu.SemaphoreType.DMA,
                       pltpu.SemaphoreType.DMA,
                       pltpu.SemaphoreType.DMA,
                       pltpu.SemaphoreType.DMA],
    )
    def k(m_h, d_h, z_h, out_h, di_v, rows0, rows1, big_v, acc_sh,
          sm0, sm1, sa0, sa1):
        cid = lax.axis_index("c")
        sid = lax.axis_index("s")
        wid = sid * _NC + cid
        for q in range(_NPT // _CH):
            r0 = sid * _NPT + q * _CH
            pltpu.sync_copy(z_h.at[pl.ds(r0, _CH)], big_v)
            pltpu.sync_copy(big_v, acc_sh.at[pl.ds(r0, _CH)])
        pltpu.sync_copy(d_h.at[wid], di_v)
        plsc.subcore_barrier()

        base = wid * _EPW
        rows = (rows0, rows1)
        sm = (sm0, sm1)
        sa = (sa0, sa1)

        def outer(o, carry):
            j = o * 2
            c0 = pltpu.async_copy(
                m_h.at[pl.ds(base + j * _CH, _CH)], rows[0], sm[0])
            c1 = pltpu.async_copy(
                m_h.at[pl.ds(base + (j + 1) * _CH, _CH)], rows[1], sm[1])
            c0.wait()
            a0 = pltpu.async_copy(rows[0], acc_sh.at[di_v.at[j]], sa[0],
                                  add=True)
            c1.wait()
            a1 = pltpu.async_copy(rows[1], acc_sh.at[di_v.at[j + 1]], sa[1],
                                  add=True)
            a0.wait()
            a1.wait()
            return carry

        lax.fori_loop(0, _NCH // 2, outer, 0, unroll=False)
        jl = _NCH - 1
        pltpu.sync_copy(m_h.at[pl.ds(base + jl * _CH, _CH)], rows0)
        pltpu.sync_copy(rows0, acc_sh.at[di_v.at[jl]], add=True)
        plsc.subcore_barrier()
        for q in range(_NPT // _CH):
            r0 = sid * _NPT + q * _CH
            pltpu.sync_copy(acc_sh.at[pl.ds(r0, _CH)], big_v)
            pltpu.sync_copy(big_v, out_h.at[cid, pl.ds(r0, _CH)])

    return k(msg, dst3, zeros)


def _dot(a, b):
    return jnp.dot(a, b, preferred_element_type=jnp.float32)


def _ln(x, g, b):
    m = jnp.mean(x, axis=-1, keepdims=True)
    v = jnp.mean((x - m) ** 2, axis=-1, keepdims=True)
    return (x - m) * lax.rsqrt(v + _EPS) * g + b


def _sp(x):
    return jnp.maximum(x, 0.0) + jnp.log1p(jnp.exp(-jnp.abs(x)))


def _row(i):
    return (i, 0)


def _rep(i):
    return (0, 0)


def _wspecs(ws):
    return [pl.BlockSpec(w.shape, _rep) for w in ws]


def _enc_node(x, typ, ws):
    def body(x_r, t_r, w0, b0, w1, b1, w2, b2, g, bl, w1d, bm1, w1s,
             q0, c0, q1, c1, q2, c2, n_o, tb_o, gm_o):
        h = jnp.maximum(_dot(x_r[...], w0[...]) + b0[...], 0.0)
        h = jnp.maximum(_dot(h, w1[...]) + b1[...], 0.0)
        n = _ln(_dot(h, w2[...]) + b2[...], g[...], bl[...])
        n_o[...] = n
        tb_o[...] = jnp.concatenate(
            [_dot(n, w1d[...]) + bm1[...], _dot(n, w1s[...])], axis=1)
        t = _sp(_dot(t_r[...], q0[...]) + c0[...])
        t = _sp(_dot(t, q1[...]) + c1[...])
        gm_o[...] = _sp(_dot(t, q2[...]) + c2[...])

    return pl.pallas_call(
        body,
        grid=(_GN,),
        in_specs=[pl.BlockSpec((_BN, x.shape[1]), _row),
                  pl.BlockSpec((_BN, typ.shape[1]), _row)] + _wspecs(ws),
        out_specs=[pl.BlockSpec((_BN, _LAT), _row),
                   pl.BlockSpec((_BN, 2 * _LAT), _row),
                   pl.BlockSpec((_BN, 1), _row)],
        out_shape=[jax.ShapeDtypeStruct((_N, _LAT), jnp.float32),
                   jax.ShapeDtypeStruct((_NP, 2 * _LAT), jnp.float32),
                   jax.ShapeDtypeStruct((_N, 1), jnp.float32)],
    )(x, typ, *ws)


def _enc_edge(ea, ws):
    grid = _E // _BE

    def body(e_r, w0, b0, w1, b1, w2, b2, g, bl, out_r):
        h = jnp.maximum(_dot(e_r[...], w0[...]) + b0[...], 0.0)
        h = jnp.maximum(_dot(h, w1[...]) + b1[...], 0.0)
        out_r[...] = _ln(_dot(h, w2[...]) + b2[...], g[...], bl[...])

    return pl.pallas_call(
        body,
        grid=(grid,),
        in_specs=[pl.BlockSpec((_BE, ea.shape[1]), _row)] + _wspecs(ws),
        out_specs=pl.BlockSpec((_BE, _LAT), _row),
        out_shape=jax.ShapeDtypeStruct((_E, _LAT), jnp.float32),
    )(ea, *ws)


def _edge_step(gsum, edge, ws):
    grid = _E // _BE

    def body(g_r, e_r, w1e, w2, b2, w3, b3, g, bl, msg_o, eo_o):
        e = e_r[...]
        h = jnp.maximum(g_r[...] + _dot(e, w1e[...]), 0.0)
        h = jnp.maximum(_dot(h, w2[...]) + b2[...], 0.0)
        msg = _ln(_dot(h, w3[...]) + b3[...], g[...], bl[...])
        msg_o[...] = jnp.concatenate([msg, jnp.zeros_like(msg)], axis=1)
        eo_o[...] = e + msg

    return pl.pallas_call(
        body,
        grid=(grid,),
        in_specs=[pl.BlockSpec((_BE, _LAT), _row)] * 2 + _wspecs(ws),
        out_specs=[pl.BlockSpec((_BE, 2 * _LAT), _row),
                   pl.BlockSpec((_BE, _LAT), _row)],
        out_shape=[jax.ShapeDtypeStruct((_E, 2 * _LAT), jnp.float32),
                   jax.ShapeDtypeStruct((_E, _LAT), jnp.float32)],
    )(gsum, edge, *ws)


def _node_body(n_r, a_r, u1n, u1a, b1, u2, b2, u3, b3, g, bl):
    n = n_r[...]
    a = a_r[...]
    agg = a[0, :, :_LAT] + a[1, :, :_LAT]
    h = jnp.maximum(_dot(n, u1n[...]) + _dot(agg, u1a[...]) + b1[...], 0.0)
    h = jnp.maximum(_dot(h, u2[...]) + b2[...], 0.0)
    upd = _ln(_dot(h, u3[...]) + b3[...], g[...], bl[...])
    return n + upd


_AGG_SPEC = pl.BlockSpec((2, _BN, 2 * _LAT), lambda i: (0, i, 0))


def _node_step(node, aggp, ws):
    def body(n_r, a_r, u1n, u1a, b1, u2, b2, u3, b3, g, bl,
             w1d, bm1, w1s, n_o, tb_o):
        nn = _node_body(n_r, a_r, u1n, u1a, b1, u2, b2, u3, b3, g, bl)
        n_o[...] = nn
        tb_o[...] = jnp.concatenate(
            [_dot(nn, w1d[...]) + bm1[...], _dot(nn, w1s[...])], axis=1)

    return pl.pallas_call(
        body,
        grid=(_GN,),
        in_specs=[pl.BlockSpec((_BN, _LAT), _row), _AGG_SPEC] + _wspecs(ws),
        out_specs=[pl.BlockSpec((_BN, _LAT), _row),
                   pl.BlockSpec((_BN, 2 * _LAT), _row)],
        out_shape=[jax.ShapeDtypeStruct((_N, _LAT), jnp.float32),
                   jax.ShapeDtypeStruct((_NP, 2 * _LAT), jnp.float32)],
    )(node, aggp, *ws)


def _decoder(node, ws):
    def body(n_r, d0, e0, d1, e1, d2, e2, m0, c0, m1, c1, m2, c2, f_o):
        o = jnp.maximum(_dot(n_r[...], d0[...]) + e0[...], 0.0)
        o = jnp.maximum(_dot(o, d1[...]) + e1[...], 0.0)
        o = _dot(o, d2[...]) + e2[...]
        f = _sp(_dot(o, m0[...]) + c0[...])
        f = _sp(_dot(f, m1[...]) + c1[...])
        f_o[...] = _dot(f, m2[...]) + c2[...]

    return pl.pallas_call(
        body,
        grid=(_GN,),
        in_specs=[pl.BlockSpec((_BN, _LAT), _row)] + _wspecs(ws),
        out_specs=pl.BlockSpec((_BN, 3), _row),
        out_shape=jax.ShapeDtypeStruct((_N, 3), jnp.float32),
    )(node, *ws)


def _b(v):
    return v.reshape(1, -1)


def _mlp_ws(ps):
    out = []
    for w, b in ps:
        out.extend([w, _b(b)])
    return out


def _split_w1(st):
    w1, b1 = st["edge_fn"][0]
    return w1[:_LAT], _b(b1), w1[_LAT:2 * _LAT], w1[2 * _LAT:]


def kernel(x, edge_index, edge_attr, type, params):
    p = params
    src = edge_index[0].astype(jnp.int32)
    dst = edge_index[1].astype(jnp.int32)
    dst3 = dst.reshape(_NW, _NCH, _CH)
    src3 = src.reshape(_NW, _NCH, _CH)
    zeros = jnp.zeros((_NP, 2 * _LAT), jnp.float32)

    steps = p["steps"]
    w1d0, b10, w1s0, _ = _split_w1(steps[0])

    enc_node_ws = (_mlp_ws(p["node_enc"])
                   + [_b(p["node_enc_ln"][0]), _b(p["node_enc_ln"][1])]
                   + [w1d0, b10, w1s0]
                   + _mlp_ws(p["mlp2"]))
    node, tbl, gamma = _enc_node(x, type, enc_node_ws)

    enc_edge_ws = (_mlp_ws(p["edge_enc"])
                   + [_b(p["edge_enc_ln"][0]), _b(p["edge_enc_ln"][1])])
    edge = _enc_edge(edge_attr, enc_edge_ws)

    def per_step(t):
        st = steps[t]
        w1d, bm1, w1s, w1e = _split_w1(st)
        del w1d, bm1, w1s
        ew = ([w1e] + _mlp_ws(st["edge_fn"][1:])
              + [_b(st["edge_ln"][0]), _b(st["edge_ln"][1])])
        un, bn = st["node_fn"][0]
        nw = ([un[:_LAT], un[_LAT:], _b(bn)] + _mlp_ws(st["node_fn"][1:])
              + [_b(st["node_ln"][0]), _b(st["node_ln"][1])])
        nst = steps[(t + 1) % len(steps)]
        w1dn, b1n, w1sn, _ = _split_w1(nst)
        return ew + nw + [w1dn, b1n, w1sn]

    n_ew = 7
    cols = list(zip(*[per_step(t) for t in range(len(steps))]))
    xs = [jnp.stack(c) for c in cols]

    def step_fn(carry, ws):
        node, tbl, edge = carry
        gsum = _sc_gather(tbl, dst3, src3)
        msg, edge = _edge_step(gsum, edge, ws[:n_ew])
        aggp = _sc_scatter(msg, dst3, zeros)
        node, tbl = _node_step(node, aggp, ws[n_ew:])
        return (node, tbl, edge), None

    (node, tbl, edge), _ = lax.scan(step_fn, (node, tbl, edge), xs)

    force = _decoder(node, _mlp_ws(p["decoder"]) + _mlp_ws(p["mlp1"]))
    return (force, gamma)


# speedup vs baseline: 2.8194x; 1.1338x over previous
import functools

import jax
import jax.numpy as jnp
from jax import lax
from jax.experimental import pallas as pl
from jax.experimental.pallas import tpu as pltpu
from jax.experimental.pallas import tpu_sc as plsc

_N = 10000
_E = 320000
_LAT = 64
_EPS = 1e-5

_NC, _NS = 2, 16
_NW = _NC * _NS
_EPW = _E // _NW
_CH = 80
_NCH = _EPW // _CH
_NP = 10240
_NPT = _NP // _NS

_BN = 2048
_GN = 5
_BE = 4000


_STG = 320


def _sc_gather(tbl, dst3, src3):
    mesh = plsc.VectorSubcoreMesh(core_axis_name="c", subcore_axis_name="s")

    @functools.partial(
        pl.kernel, mesh=mesh,
        out_type=jax.ShapeDtypeStruct((_E, _LAT), jnp.float32),
        scratch_types=[pltpu.VMEM((_NCH, _CH), jnp.int32),
                       pltpu.VMEM((_NCH, _CH), jnp.int32),
                       pltpu.VMEM((_CH, 2 * _LAT), jnp.float32),
                       pltpu.VMEM((_CH, 2 * _LAT), jnp.float32),
                       pltpu.VMEM((_CH, 2 * _LAT), jnp.float32),
                       pltpu.VMEM((_CH, 2 * _LAT), jnp.float32),
                       pltpu.VMEM((_CH, _LAT), jnp.float32),
                       pltpu.VMEM((_CH, _LAT), jnp.float32),
                       pltpu.SemaphoreType.DMA,
                       pltpu.SemaphoreType.DMA,
                       pltpu.SemaphoreType.DMA,
                       pltpu.SemaphoreType.DMA,
                       pltpu.SemaphoreType.DMA],
    )
    def k(t_h, d_h, s_h, g_h, di_v, si_v, rd0, rs0, rd1, rs1, g0, g1,
          sd0, ss0, sd1, ss1, sw):
        sid = lax.axis_index("s")
        cid = lax.axis_index("c")
        wid = sid * _NC + cid
        pltpu.sync_copy(d_h.at[wid], di_v)
        pltpu.sync_copy(s_h.at[wid], si_v)

        base = wid * _EPW
        rd = (rd0, rd1)
        rs = (rs0, rs1)
        gg = (g0, g1)
        sd = (sd0, sd1)
        ss = (ss0, ss1)

        def add_rows(rd_v, rs_v, g_v):
            @plsc.parallel_loop(0, _CH, unroll=8)
            def add_row(r):
                for c in range(_LAT // 16):
                    g_v[r, pl.ds(c * 16, 16)] = (
                        rd_v[r, pl.ds(c * 16, 16)]
                        + rs_v[r, pl.ds(_LAT + c * 16, 16)])

        def do_chunk(j, b, cpd, cps):
            cpd.wait()
            cps.wait()
            add_rows(rd[b], rs[b], gg[b])
            return pltpu.async_copy(
                gg[b], g_h.at[pl.ds(base + j * _CH, _CH)], sw)

        def outer(o, carry):
            j = o * 2
            cd0 = pltpu.async_copy(t_h.at[di_v.at[j]], rd[0], sd[0])
            cs0 = pltpu.async_copy(t_h.at[si_v.at[j]], rs[0], ss[0])
            cd1 = pltpu.async_copy(t_h.at[di_v.at[j + 1]], rd[1], sd[1])
            cs1 = pltpu.async_copy(t_h.at[si_v.at[j + 1]], rs[1], ss[1])
            w0 = do_chunk(j, 0, cd0, cs0)
            w1 = do_chunk(j + 1, 1, cd1, cs1)
            w0.wait()
            w1.wait()
            return carry

        lax.fori_loop(0, _NCH // 2, outer, 0, unroll=False)
        jl = _NCH - 1
        cd = pltpu.async_copy(t_h.at[di_v.at[jl]], rd[0], sd[0])
        cs = pltpu.async_copy(t_h.at[si_v.at[jl]], rs[0], ss[0])
        do_chunk(jl, 0, cd, cs).wait()

    return k(tbl, dst3, src3)


def _sc_scatter(msg, dst3, zeros):
    mesh = plsc.VectorSubcoreMesh(core_axis_name="c", subcore_axis_name="s")

    @functools.partial(
        pl.kernel, mesh=mesh,
        out_type=jax.ShapeDtypeStruct((_NC, _NP, 2 * _LAT), jnp.float32),
        scratch_types=[pltpu.VMEM((_NCH, _CH), jnp.int32),
                       pltpu.VMEM((_CH, 2 * _LAT), jnp.float32),
                       pltpu.VMEM((_CH, 2 * _LAT), jnp.float32),
                       pltpu.VMEM((_CH, 2 * _LAT), jnp.float32),
                       pltpu.VMEM_SHARED((_NP, 2 * _LAT), jnp.float32),
                       pltpu.SemaphoreType.DMA,
                       pltpu.SemaphoreType.DMA,
                       pltpu.SemaphoreType.DMA,
                       pltpu.SemaphoreType.DMA],
    )
    def k(m_h, d_h, z_h, out_h, di_v, rows0, rows1, big_v, acc_sh,
          sm0, sm1, sa0, sa1):
        cid = lax.axis_index("c")
        sid = lax.axis_index("s")
        wid = sid * _NC + cid
        for q in range(_NPT // _CH):
            r0 = sid * _NPT + q * _CH
            pltpu.sync_copy(z_h.at[pl.ds(r0, _CH)], big_v)
            pltpu.sync_copy(big_v, acc_sh.at[pl.ds(r0, _CH)])
        pltpu.sync_copy(d_h.at[wid], di_v)
        plsc.subcore_barrier()

        base = wid * _EPW
        rows = (rows0, rows1)
        sm = (sm0, sm1)
        sa = (sa0, sa1)

        def outer(o, carry):
            j = o * 2
            c0 = pltpu.async_copy(
                m_h.at[pl.ds(base + j * _CH, _CH)], rows[0], sm[0])
            c1 = pltpu.async_copy(
                m_h.at[pl.ds(base + (j + 1) * _CH, _CH)], rows[1], sm[1])
            c0.wait()
            a0 = pltpu.async_copy(rows[0], acc_sh.at[di_v.at[j]], sa[0],
                                  add=True)
            c1.wait()
            a1 = pltpu.async_copy(rows[1], acc_sh.at[di_v.at[j + 1]], sa[1],
                                  add=True)
            a0.wait()
            a1.wait()
            return carry

        lax.fori_loop(0, _NCH // 2, outer, 0, unroll=False)
        jl = _NCH - 1
        pltpu.sync_copy(m_h.at[pl.ds(base + jl * _CH, _CH)], rows0)
        pltpu.sync_copy(rows0, acc_sh.at[di_v.at[jl]], add=True)
        plsc.subcore_barrier()
        for q in range(_NPT // _CH):
            r0 = sid * _NPT + q * _CH
            pltpu.sync_copy(acc_sh.at[pl.ds(r0, _CH)], big_v)
            pltpu.sync_copy(big_v, out_h.at[cid, pl.ds(r0, _CH)])

    return k(msg, dst3, zeros)


def _dot(a, b):
    return jnp.dot(a, b, preferred_element_type=jnp.float32)


def _ln(x, g, b):
    m = jnp.mean(x, axis=-1, keepdims=True)
    v = jnp.mean((x - m) ** 2, axis=-1, keepdims=True)
    return (x - m) * lax.rsqrt(v + _EPS) * g + b


def _sp(x):
    return jnp.maximum(x, 0.0) + jnp.log1p(jnp.exp(-jnp.abs(x)))


def _row(i):
    return (i, 0)


def _rep(i):
    return (0, 0)


def _wspecs(ws):
    return [pl.BlockSpec(w.shape, _rep) for w in ws]


def _enc_node(x, typ, ws):
    def body(x_r, t_r, w0, b0, w1, b1, w2, b2, g, bl, w1d, bm1, w1s,
             q0, c0, q1, c1, q2, c2, n_o, tb_o, gm_o):
        h = jnp.maximum(_dot(x_r[...], w0[...]) + b0[...], 0.0)
        h = jnp.maximum(_dot(h, w1[...]) + b1[...], 0.0)
        n = _ln(_dot(h, w2[...]) + b2[...], g[...], bl[...])
        n_o[...] = n
        tb_o[...] = jnp.concatenate(
            [_dot(n, w1d[...]) + bm1[...], _dot(n, w1s[...])], axis=1)
        t = _sp(_dot(t_r[...], q0[...]) + c0[...])
        t = _sp(_dot(t, q1[...]) + c1[...])
        gm_o[...] = _sp(_dot(t, q2[...]) + c2[...])

    return pl.pallas_call(
        body,
        grid=(_GN,),
        in_specs=[pl.BlockSpec((_BN, x.shape[1]), _row),
                  pl.BlockSpec((_BN, typ.shape[1]), _row)] + _wspecs(ws),
        out_specs=[pl.BlockSpec((_BN, _LAT), _row),
                   pl.BlockSpec((_BN, 2 * _LAT), _row),
                   pl.BlockSpec((_BN, 1), _row)],
        out_shape=[jax.ShapeDtypeStruct((_N, _LAT), jnp.float32),
                   jax.ShapeDtypeStruct((_NP, 2 * _LAT), jnp.float32),
                   jax.ShapeDtypeStruct((_N, 1), jnp.float32)],
    )(x, typ, *ws)


def _enc_edge(ea, ws):
    grid = _E // _BE

    def body(e_r, w0, b0, w1, b1, w2, b2, g, bl, out_r):
        h = jnp.maximum(_dot(e_r[...], w0[...]) + b0[...], 0.0)
        h = jnp.maximum(_dot(h, w1[...]) + b1[...], 0.0)
        out_r[...] = _ln(_dot(h, w2[...]) + b2[...], g[...], bl[...])

    return pl.pallas_call(
        body,
        grid=(grid,),
        in_specs=[pl.BlockSpec((_BE, ea.shape[1]), _row)] + _wspecs(ws),
        out_specs=pl.BlockSpec((_BE, _LAT), _row),
        out_shape=jax.ShapeDtypeStruct((_E, _LAT), jnp.float32),
    )(ea, *ws)


def _edge_step(gsum, edge, ws):
    grid = _E // _BE

    def body(g_r, e_r, w1e, w2, b2, w3, b3, g, bl, msg_o, eo_o):
        e = e_r[...]
        h = jnp.maximum(g_r[...] + _dot(e, w1e[...]), 0.0)
        h = jnp.maximum(_dot(h, w2[...]) + b2[...], 0.0)
        msg = _ln(_dot(h, w3[...]) + b3[...], g[...], bl[...])
        msg_o[...] = jnp.concatenate([msg, jnp.zeros_like(msg)], axis=1)
        eo_o[...] = e + msg

    return pl.pallas_call(
        body,
        grid=(grid,),
        in_specs=[pl.BlockSpec((_BE, _LAT), _row)] * 2 + _wspecs(ws),
        out_specs=[pl.BlockSpec((_BE, 2 * _LAT), _row),
                   pl.BlockSpec((_BE, _LAT), _row)],
        out_shape=[jax.ShapeDtypeStruct((_E, 2 * _LAT), jnp.float32),
                   jax.ShapeDtypeStruct((_E, _LAT), jnp.float32)],
    )(gsum, edge, *ws)


def _node_body(n_r, a_r, u1n, u1a, b1, u2, b2, u3, b3, g, bl):
    n = n_r[...]
    a = a_r[...]
    agg = a[0, :, :_LAT] + a[1, :, :_LAT]
    h = jnp.maximum(_dot(n, u1n[...]) + _dot(agg, u1a[...]) + b1[...], 0.0)
    h = jnp.maximum(_dot(h, u2[...]) + b2[...], 0.0)
    upd = _ln(_dot(h, u3[...]) + b3[...], g[...], bl[...])
    return n + upd


_AGG_SPEC = pl.BlockSpec((2, _BN, 2 * _LAT), lambda i: (0, i, 0))


def _node_step(node, aggp, ws):
    def body(n_r, a_r, u1n, u1a, b1, u2, b2, u3, b3, g, bl,
             w1d, bm1, w1s, n_o, tb_o):
        nn = _node_body(n_r, a_r, u1n, u1a, b1, u2, b2, u3, b3, g, bl)
        n_o[...] = nn
        tb_o[...] = jnp.concatenate(
            [_dot(nn, w1d[...]) + bm1[...], _dot(nn, w1s[...])], axis=1)

    return pl.pallas_call(
        body,
        grid=(_GN,),
        in_specs=[pl.BlockSpec((_BN, _LAT), _row), _AGG_SPEC] + _wspecs(ws),
        out_specs=[pl.BlockSpec((_BN, _LAT), _row),
                   pl.BlockSpec((_BN, 2 * _LAT), _row)],
        out_shape=[jax.ShapeDtypeStruct((_N, _LAT), jnp.float32),
                   jax.ShapeDtypeStruct((_NP, 2 * _LAT), jnp.float32)],
    )(node, aggp, *ws)


def _decoder(node, ws):
    def body(n_r, d0, e0, d1, e1, d2, e2, m0, c0, m1, c1, m2, c2, f_o):
        o = jnp.maximum(_dot(n_r[...], d0[...]) + e0[...], 0.0)
        o = jnp.maximum(_dot(o, d1[...]) + e1[...], 0.0)
        o = _dot(o, d2[...]) + e2[...]
        f = _sp(_dot(o, m0[...]) + c0[...])
        f = _sp(_dot(f, m1[...]) + c1[...])
        f_o[...] = _dot(f, m2[...]) + c2[...]

    return pl.pallas_call(
        body,
        grid=(_GN,),
        in_specs=[pl.BlockSpec((_BN, _LAT), _row)] + _wspecs(ws),
        out_specs=pl.BlockSpec((_BN, 3), _row),
        out_shape=jax.ShapeDtypeStruct((_N, 3), jnp.float32),
    )(node, *ws)


def _b(v):
    return v.reshape(1, -1)


def _mlp_ws(ps):
    out = []
    for w, b in ps:
        out.extend([w, _b(b)])
    return out


def _split_w1(st):
    w1, b1 = st["edge_fn"][0]
    return w1[:_LAT], _b(b1), w1[_LAT:2 * _LAT], w1[2 * _LAT:]


def kernel(x, edge_index, edge_attr, type, params):
    p = params
    src = edge_index[0].astype(jnp.int32)
    dst = edge_index[1].astype(jnp.int32)
    dst3 = dst.reshape(_NW, _NCH, _CH)
    src3 = src.reshape(_NW, _NCH, _CH)
    zeros = jnp.zeros((_NP, 2 * _LAT), jnp.float32)

    steps = p["steps"]
    w1d0, b10, w1s0, _ = _split_w1(steps[0])

    enc_node_ws = (_mlp_ws(p["node_enc"])
                   + [_b(p["node_enc_ln"][0]), _b(p["node_enc_ln"][1])]
                   + [w1d0, b10, w1s0]
                   + _mlp_ws(p["mlp2"]))
    node, tbl, gamma = _enc_node(x, type, enc_node_ws)

    enc_edge_ws = (_mlp_ws(p["edge_enc"])
                   + [_b(p["edge_enc_ln"][0]), _b(p["edge_enc_ln"][1])])
    edge = _enc_edge(edge_attr, enc_edge_ws)

    def per_step(t):
        st = steps[t]
        w1d, bm1, w1s, w1e = _split_w1(st)
        del w1d, bm1, w1s
        ew = ([w1e] + _mlp_ws(st["edge_fn"][1:])
              + [_b(st["edge_ln"][0]), _b(st["edge_ln"][1])])
        un, bn = st["node_fn"][0]
        nw = ([un[:_LAT], un[_LAT:], _b(bn)] + _mlp_ws(st["node_fn"][1:])
              + [_b(st["node_ln"][0]), _b(st["node_ln"][1])])
        nst = steps[(t + 1) % len(steps)]
        w1dn, b1n, w1sn, _ = _split_w1(nst)
        return ew + nw + [w1dn, b1n, w1sn]

    n_ew = 7
    cols = list(zip(*[per_step(t) for t in range(len(steps))]))
    xs = [jnp.stack(c) for c in cols]

    def step_fn(carry, ws):
        node, tbl, edge = carry
        gsum = _sc_gather(tbl, dst3, src3)
        msg, edge = _edge_step(gsum, edge, ws[:n_ew])
        aggp = _sc_scatter(msg, dst3, zeros)
        node, tbl = _node_step(node, aggp, ws[n_ew:])
        return (node, tbl, edge), None

    (node, tbl, edge), _ = lax.scan(step_fn, (node, tbl, edge), xs)

    force = _decoder(node, _mlp_ws(p["decoder"]) + _mlp_ws(p["mlp1"]))
    return (force, gamma)
